# baseline (speedup 1.0000x reference)
import jax
import jax.numpy as jnp
from jax import lax
from jax.experimental import pallas as pl
from jax.experimental.pallas import tpu as pltpu

_BQ = 4096


def _block(x_ref, p_ref, out_ref):
    x = x_ref[...]
    p = p_ref[...]
    n_out = out_ref.shape[1]
    x2 = jnp.sum(x * x, axis=1, keepdims=True)
    p2 = jnp.sum(p * p, axis=1)[None, :]
    dot = lax.dot_general(
        x, p, (((1,), (1,)), ((), ())),
        preferred_element_type=jnp.float32)
    d2 = x2 + p2 - 2.0 * dot
    lab = jnp.argmin(d2, axis=1).astype(jnp.int32) + 1
    cls = lax.broadcasted_iota(jnp.int32, (_BQ, n_out), 1)
    out_ref[...] = (cls == lab[:, None]).astype(jnp.float32)


def kernel(x, protos):
    q, d = x.shape
    k, _ = protos.shape
    n_out = k + 1
    ni = q // _BQ
    out = pl.pallas_call(
        _block,
        grid=(ni,),
        in_specs=[
            pl.BlockSpec((_BQ, d), lambda i: (i, 0)),
            pl.BlockSpec((k, d), lambda i: (0, 0)),
        ],
        out_specs=pl.BlockSpec((_BQ, n_out), lambda i: (i, 0)),
        out_shape=jax.ShapeDtypeStruct((q, n_out), jnp.float32),
        compiler_params=pltpu.CompilerParams(
            dimension_semantics=("parallel",)),
    )(x, protos)
    return out


# speedup vs baseline: 1.0110x; 1.0110x over previous
import jax
import jax.numpy as jnp
from jax import lax
from jax.experimental import pallas as pl
from jax.experimental.pallas import tpu as pltpu

_BQ = 2048


def _block(x_ref, p_ref, out_ref):
    x = x_ref[...]
    p = p_ref[...]
    n_out = out_ref.shape[1]
    x2 = jnp.sum(x * x, axis=1, keepdims=True)
    p2 = jnp.sum(p * p, axis=1)[None, :]
    dot = lax.dot_general(
        x, p, (((1,), (1,)), ((), ())),
        preferred_element_type=jnp.float32)
    d2 = x2 + p2 - 2.0 * dot
    lab = jnp.argmin(d2, axis=1).astype(jnp.int32) + 1
    cls = lax.broadcasted_iota(jnp.int32, (_BQ, n_out), 1)
    out_ref[...] = (cls == lab[:, None]).astype(jnp.float32)


def kernel(x, protos):
    q, d = x.shape
    k, _ = protos.shape
    n_out = k + 1
    ni = q // _BQ
    out = pl.pallas_call(
        _block,
        grid=(ni,),
        in_specs=[
            pl.BlockSpec((_BQ, d), lambda i: (i, 0)),
            pl.BlockSpec((k, d), lambda i: (0, 0)),
        ],
        out_specs=pl.BlockSpec((_BQ, n_out), lambda i: (i, 0)),
        out_shape=jax.ShapeDtypeStruct((q, n_out), jnp.float32),
        compiler_params=pltpu.CompilerParams(
            dimension_semantics=("parallel",)),
    )(x, protos)
    return out
